# baseline (speedup 1.0000x reference)
import jax
import jax.numpy as jnp
from jax import lax
from jax.experimental import pallas as pl
from jax.experimental.pallas import tpu as pltpu
from jax.experimental.pallas import tpu_sc as plsc

H = 2048
W = 2048
C = 16
M = 1048576

NC = 2
NS = 16
NW = NC * NS

_F = jnp.float32
_I = jnp.int32

TY = H // 8
TX = W // 128
NT = TY * TX
TPW = NT // NW

BP = 128
CHUNK = M // NW
NBK = CHUNK // BP
GPB = BP // 16
NBLK_ALL = M // BP


def _tr_body(grid_hbm, table_hbm, ch0, ch1, ot0, ot1, si0, si1, so0, so1):
    chs = (ch0, ch1)
    ots = (ot0, ot1)
    sis = (si0, si1)
    sos = (so0, so1)
    wid = lax.axis_index("s") * NC + lax.axis_index("c")
    base = wid * TPW
    iota = lax.iota(_I, 16)
    cvecs = [jnp.full((16,), c, dtype=_I) for c in range(C)]

    def fire_in(par, b):
        t = base + jnp.minimum(b, TPW - 1)
        ty = t // TX
        tx = t - ty * TX
        for c in range(C):
            pltpu.async_copy(grid_hbm.at[c, ty, tx], chs[par].at[c], sis[par])

    def wait_in(par):
        for c in range(C):
            pltpu.make_async_copy(grid_hbm.at[c, 0, 0], chs[par].at[c],
                                  sis[par]).wait()

    def interleave(par):
        def grp(k):
            iy = k // 8
            j = k - iy * 8
            pidx = iy * 128 + j * 16 + iota
            for c in range(C):
                v = chs[par][c, iy, pl.ds(j * 16, 16)]
                plsc.store_scatter(ots[par], [pidx, cvecs[c]], v)

        plsc.parallel_loop(0, 64, 1, unroll=8)(grp)

    def fire_out(par, b):
        t = base + b
        ty = t // TX
        tx = t - ty * TX
        for iy in range(8):
            row = (ty * 8 + iy) * W + tx * 128
            pltpu.async_copy(ots[par].at[pl.ds(iy * 128, 128)],
                             table_hbm.at[pl.ds(row, 128)], sos[par])

    def wait_out(par):
        for iy in range(8):
            pltpu.make_async_copy(ots[par].at[pl.ds(iy * 128, 128)],
                                  table_hbm.at[pl.ds(0, 128)],
                                  sos[par]).wait()

    fire_in(0, 0)

    def sup(it, carry):
        for par in (0, 1):
            b = 2 * it + par
            wait_in(par)
            fire_in(1 - par, b + 1)

            @pl.when(b >= 2)
            def _():
                wait_out(par)

            interleave(par)
            fire_out(par, b)
        return carry

    lax.fori_loop(0, TPW // 2, sup, 0)
    wait_in(0)
    wait_out(0)
    wait_out(1)


def _sa_body(table_hbm, coords_hbm, out_hbm,
             co0, co1,
             a0, b0, c0, d0, a1, b1, c1, d1,
             wx0, wy0, wx1, wy1,
             r00_0, r01_0, r10_0, r11_0, r00_1, r01_1, r10_1, r11_1,
             o0, o1,
             sc0, sc1, sg0, sg1, so0, so1):
    cos = (co0, co1)
    idxs = ((a0, b0, c0, d0), (a1, b1, c1, d1))
    ws = ((wx0, wy0), (wx1, wy1))
    rows = ((r00_0, r01_0, r10_0, r11_0), (r00_1, r01_1, r10_1, r11_1))
    outs = (o0, o1)
    scs = (sc0, sc1)
    sgs = (sg0, sg1)
    sos = (so0, so1)

    wid = lax.axis_index("s") * NC + lax.axis_index("c")
    bbase = wid * NBK
    iota = lax.iota(_I, 16)
    cvecs = [jnp.full((16,), c, dtype=_I) for c in range(C)]

    def fire_co(par, g):
        bid = bbase + jnp.minimum(g, NBK - 1)
        pltpu.async_copy(coords_hbm.at[bid], cos[par], scs[par])

    def wait_co(par):
        pltpu.make_async_copy(coords_hbm.at[0], cos[par], scs[par]).wait()

    def pass1(par):
        i00_v, i01_v, i10_v, i11_v = idxs[par]
        wx_v, wy_v = ws[par]

        def grp(j):
            sl = pl.ds(j * 16, 16)
            x = cos[par][0, sl]
            y = cos[par][1, sl]
            u = 2.0 * x - 1.0
            v = 2.0 * y - 1.0
            xp = (u + 1.0) * 0.5 * (W - 1)
            yp = (v + 1.0) * 0.5 * (H - 1)
            xp = jnp.minimum(jnp.maximum(xp, 0.0), float(W - 1))
            yp = jnp.minimum(jnp.maximum(yp, 0.0), float(H - 1))
            x0i = xp.astype(_I)
            y0i = yp.astype(_I)
            wx_v[sl] = xp - x0i.astype(_F)
            wy_v[sl] = yp - y0i.astype(_F)
            x1i = jnp.minimum(x0i + 1, W - 1)
            y1i = jnp.minimum(y0i + 1, H - 1)
            y0w = y0i * W
            y1w = y1i * W
            i00_v[sl] = y0w + x0i
            i01_v[sl] = y0w + x1i
            i10_v[sl] = y1w + x0i
            i11_v[sl] = y1w + x1i

        plsc.parallel_loop(0, GPB, 1, unroll=GPB)(grp)

    def fire_g(par):
        for i in range(4):
            pltpu.async_copy(table_hbm.at[idxs[par][i]], rows[par][i],
                             sgs[par])

    def wait_g(par):
        for i in range(4):
            pltpu.make_async_copy(table_hbm.at[idxs[par][i]], rows[par][i],
                                  sgs[par]).wait()

    def pass2(par):
        r00_v, r01_v, r10_v, r11_v = rows[par]
        wx_v, wy_v = ws[par]

        def grp(j):
            sl = pl.ds(j * 16, 16)
            pidx = j * 16 + iota
            wx = wx_v[sl]
            wy = wy_v[sl]
            w11 = wx * wy
            w10 = wy - w11
            w01 = wx - w11
            w00 = (1.0 - wx) - w10
            for c in range(C):
                ia = plsc.load_gather(r00_v, [pidx, cvecs[c]])
                ib = plsc.load_gather(r01_v, [pidx, cvecs[c]])
                ic = plsc.load_gather(r10_v, [pidx, cvecs[c]])
                id_ = plsc.load_gather(r11_v, [pidx, cvecs[c]])
                outs[par][c, sl] = ia * w00 + ib * w01 + ic * w10 + id_ * w11

        plsc.parallel_loop(0, GPB, 1, unroll=GPB)(grp)

    def fire_o(par, g):
        bid = bbase + g
        pltpu.async_copy(outs[par].at[pl.ds(0, 8)], out_hbm.at[0, bid],
                         sos[par])
        pltpu.async_copy(outs[par].at[pl.ds(8, 8)], out_hbm.at[1, bid],
                         sos[par])

    def wait_o(par):
        pltpu.make_async_copy(outs[par].at[pl.ds(0, 8)], out_hbm.at[0, 0],
                              sos[par]).wait()
        pltpu.make_async_copy(outs[par].at[pl.ds(8, 8)], out_hbm.at[1, 0],
                              sos[par]).wait()

    fire_co(0, 0)
    wait_co(0)
    pass1(0)
    fire_g(0)
    fire_co(1, 1)

    def sup(it, carry):
        for par in (0, 1):
            g = 2 * it + par
            wait_g(par)
            wait_co(1 - par)
            pass1(1 - par)
            fire_g(1 - par)
            fire_co(par, g + 2)

            @pl.when(g >= 2)
            def _():
                wait_o(par)

            pass2(par)
            fire_o(par, g)
        return carry

    lax.fori_loop(0, NBK // 2, sup, 0)
    wait_g(0)
    wait_co(1)
    wait_o(0)
    wait_o(1)


@jax.jit
def _grid_sample_sc(grid5, coords3):
    mesh = plsc.VectorSubcoreMesh(core_axis_name="c", subcore_axis_name="s")
    params = pltpu.CompilerParams(
        needs_layout_passes=False, use_tc_tiling_on_sc=False
    )
    table = pl.kernel(
        _tr_body,
        out_type=jax.ShapeDtypeStruct((H * W, C), _F),
        mesh=mesh,
        scratch_types=[
            pltpu.VMEM((C, 8, 128), _F),
            pltpu.VMEM((C, 8, 128), _F),
            pltpu.VMEM((1024, C), _F),
            pltpu.VMEM((1024, C), _F),
            pltpu.SemaphoreType.DMA,
            pltpu.SemaphoreType.DMA,
            pltpu.SemaphoreType.DMA,
            pltpu.SemaphoreType.DMA,
        ],
        compiler_params=params,
    )(grid5)

    return pl.kernel(
        _sa_body,
        out_type=jax.ShapeDtypeStruct((2, NBLK_ALL, 8, BP), _F),
        mesh=mesh,
        scratch_types=[
            pltpu.VMEM((2, BP), _F),
            pltpu.VMEM((2, BP), _F),
            pltpu.VMEM((BP,), _I),
            pltpu.VMEM((BP,), _I),
            pltpu.VMEM((BP,), _I),
            pltpu.VMEM((BP,), _I),
            pltpu.VMEM((BP,), _I),
            pltpu.VMEM((BP,), _I),
            pltpu.VMEM((BP,), _I),
            pltpu.VMEM((BP,), _I),
            pltpu.VMEM((BP,), _F),
            pltpu.VMEM((BP,), _F),
            pltpu.VMEM((BP,), _F),
            pltpu.VMEM((BP,), _F),
            pltpu.VMEM((BP, C), _F),
            pltpu.VMEM((BP, C), _F),
            pltpu.VMEM((BP, C), _F),
            pltpu.VMEM((BP, C), _F),
            pltpu.VMEM((BP, C), _F),
            pltpu.VMEM((BP, C), _F),
            pltpu.VMEM((BP, C), _F),
            pltpu.VMEM((BP, C), _F),
            pltpu.VMEM((C, BP), _F),
            pltpu.VMEM((C, BP), _F),
            pltpu.SemaphoreType.DMA,
            pltpu.SemaphoreType.DMA,
            pltpu.SemaphoreType.DMA,
            pltpu.SemaphoreType.DMA,
            pltpu.SemaphoreType.DMA,
            pltpu.SemaphoreType.DMA,
        ],
        compiler_params=params,
    )(table, coords3)


def kernel(coords_local_xy, w_grid):
    coords = coords_local_xy.astype(jnp.float32)
    grid5 = jnp.transpose(
        jnp.reshape(w_grid[0], (C, TY, 8, TX, 128)), (0, 1, 3, 2, 4)
    )
    coords3 = jnp.transpose(
        jnp.reshape(coords, (NBLK_ALL, BP, 2)), (0, 2, 1)
    )
    v = _grid_sample_sc(grid5, coords3)
    return jnp.reshape(jnp.transpose(v, (1, 3, 0, 2)), (M, C))


# speedup vs baseline: 1.4213x; 1.4213x over previous
import jax
import jax.numpy as jnp
from jax import lax
from jax.experimental import pallas as pl
from jax.experimental.pallas import tpu as pltpu
from jax.experimental.pallas import tpu_sc as plsc

H = 2048
W = 2048
C = 16
M = 1048576

NC = 2
NS = 16
NW = NC * NS

_F = jnp.float32
_I = jnp.int32

TY = H // 8
TX = W // 128
NT = TY * TX
TPW = NT // NW

BP = 128
CHUNK = M // NW
NBK = CHUNK // BP
GPB = BP // 16
NBLK_ALL = M // BP


def _tr_body(grid_hbm, table_hbm, ch0, ch1, ot0, ot1, si0, si1, so0, so1):
    chs = (ch0, ch1)
    ots = (ot0, ot1)
    sis = (si0, si1)
    sos = (so0, so1)
    wid = lax.axis_index("s") * NC + lax.axis_index("c")
    base = wid * TPW
    iota = lax.iota(_I, 16)
    cvecs = [jnp.full((16,), c, dtype=_I) for c in range(C)]

    def fire_in(par, b):
        t = base + jnp.minimum(b, TPW - 1)
        ty = t // TX
        tx = t - ty * TX
        for c in range(C):
            pltpu.async_copy(grid_hbm.at[c, ty, tx], chs[par].at[c], sis[par])

    def wait_in(par):
        for c in range(C):
            pltpu.make_async_copy(grid_hbm.at[c, 0, 0], chs[par].at[c],
                                  sis[par]).wait()

    def interleave(par):
        def grp(k):
            iy = k // 8
            j = k - iy * 8
            pidx = iy * 128 + j * 16 + iota
            for c in range(C):
                v = chs[par][c, iy, pl.ds(j * 16, 16)]
                plsc.store_scatter(ots[par], [pidx, cvecs[c]], v)

        plsc.parallel_loop(0, 64, 1, unroll=4)(grp)

    def fire_out(par, b):
        t = base + b
        ty = t // TX
        tx = t - ty * TX
        for iy in range(8):
            row = (ty * 8 + iy) * W + tx * 128
            pltpu.async_copy(ots[par].at[pl.ds(iy * 128, 128)],
                             table_hbm.at[pl.ds(row, 128)], sos[par])

    def wait_out(par):
        for iy in range(8):
            pltpu.make_async_copy(ots[par].at[pl.ds(iy * 128, 128)],
                                  table_hbm.at[pl.ds(0, 128)],
                                  sos[par]).wait()

    fire_in(0, 0)

    def sup(it, carry):
        for par in (0, 1):
            b = 2 * it + par
            wait_in(par)
            fire_in(1 - par, b + 1)

            @pl.when(b >= 2)
            def _():
                wait_out(par)

            interleave(par)
            fire_out(par, b)
        return carry

    lax.fori_loop(0, TPW // 2, sup, 0)
    wait_in(0)
    wait_out(0)
    wait_out(1)


def _sa_body(table_hbm, coords_hbm, out_hbm,
             co0, co1,
             a0, b0, c0, d0, a1, b1, c1, d1,
             wx0, wy0, wx1, wy1,
             r00_0, r01_0, r10_0, r11_0, r00_1, r01_1, r10_1, r11_1,
             o0, o1,
             sc0, sc1, sg0, sg1, so0, so1):
    cos = (co0, co1)
    idxs = ((a0, b0, c0, d0), (a1, b1, c1, d1))
    ws = ((wx0, wy0), (wx1, wy1))
    rows = ((r00_0, r01_0, r10_0, r11_0), (r00_1, r01_1, r10_1, r11_1))
    outs = (o0, o1)
    scs = (sc0, sc1)
    sgs = (sg0, sg1)
    sos = (so0, so1)

    wid = lax.axis_index("s") * NC + lax.axis_index("c")
    bbase = wid * NBK
    iota = lax.iota(_I, 16)
    cdiags = [(iota + c) % 16 for c in range(C)]

    def fire_co(par, g):
        bid = bbase + jnp.minimum(g, NBK - 1)
        pltpu.async_copy(coords_hbm.at[bid], cos[par], scs[par])

    def wait_co(par):
        pltpu.make_async_copy(coords_hbm.at[0], cos[par], scs[par]).wait()

    def pass1(par):
        i00_v, i01_v, i10_v, i11_v = idxs[par]
        wx_v, wy_v = ws[par]

        def grp(j):
            sl = pl.ds(j * 16, 16)
            x = cos[par][0, sl]
            y = cos[par][1, sl]
            u = 2.0 * x - 1.0
            v = 2.0 * y - 1.0
            xp = (u + 1.0) * 0.5 * (W - 1)
            yp = (v + 1.0) * 0.5 * (H - 1)
            xp = jnp.minimum(jnp.maximum(xp, 0.0), float(W - 1))
            yp = jnp.minimum(jnp.maximum(yp, 0.0), float(H - 1))
            x0i = xp.astype(_I)
            y0i = yp.astype(_I)
            wx_v[sl] = xp - x0i.astype(_F)
            wy_v[sl] = yp - y0i.astype(_F)
            x1i = jnp.minimum(x0i + 1, W - 1)
            y1i = jnp.minimum(y0i + 1, H - 1)
            y0w = y0i * W
            y1w = y1i * W
            i00_v[sl] = y0w + x0i
            i01_v[sl] = y0w + x1i
            i10_v[sl] = y1w + x0i
            i11_v[sl] = y1w + x1i

        plsc.parallel_loop(0, GPB, 1, unroll=GPB)(grp)

    def fire_g(par):
        for i in range(4):
            pltpu.async_copy(table_hbm.at[idxs[par][i]], rows[par][i],
                             sgs[par])

    def wait_g(par):
        for i in range(4):
            pltpu.make_async_copy(table_hbm.at[idxs[par][i]], rows[par][i],
                                  sgs[par]).wait()

    def pass2(par):
        r00_v, r01_v, r10_v, r11_v = rows[par]
        wx_v, wy_v = ws[par]

        def grp(j):
            sl = pl.ds(j * 16, 16)
            pidx = j * 16 + iota
            wx = wx_v[sl]
            wy = wy_v[sl]
            w11 = wx * wy
            w10 = wy - w11
            w01 = wx - w11
            w00 = (1.0 - wx) - w10
            for c in range(C):
                cd = cdiags[c]
                ia = plsc.load_gather(r00_v, [pidx, cd])
                ib = plsc.load_gather(r01_v, [pidx, cd])
                ic = plsc.load_gather(r10_v, [pidx, cd])
                id_ = plsc.load_gather(r11_v, [pidx, cd])
                val = ia * w00 + ib * w01 + ic * w10 + id_ * w11
                plsc.store_scatter(outs[par], [cd, pidx], val)

        plsc.parallel_loop(0, GPB, 1, unroll=4)(grp)

    def fire_o(par, g):
        bid = bbase + g
        pltpu.async_copy(outs[par].at[pl.ds(0, 8)], out_hbm.at[0, bid],
                         sos[par])
        pltpu.async_copy(outs[par].at[pl.ds(8, 8)], out_hbm.at[1, bid],
                         sos[par])

    def wait_o(par):
        pltpu.make_async_copy(outs[par].at[pl.ds(0, 8)], out_hbm.at[0, 0],
                              sos[par]).wait()
        pltpu.make_async_copy(outs[par].at[pl.ds(8, 8)], out_hbm.at[1, 0],
                              sos[par]).wait()

    fire_co(0, 0)
    wait_co(0)
    pass1(0)
    fire_g(0)
    fire_co(1, 1)

    def sup(it, carry):
        for par in (0, 1):
            g = 2 * it + par
            wait_g(par)
            wait_co(1 - par)
            pass1(1 - par)
            fire_g(1 - par)
            fire_co(par, g + 2)

            @pl.when(g >= 2)
            def _():
                wait_o(par)

            pass2(par)
            fire_o(par, g)
        return carry

    lax.fori_loop(0, NBK // 2, sup, 0)
    wait_g(0)
    wait_co(1)
    wait_o(0)
    wait_o(1)


@jax.jit
def _grid_sample_sc(grid5, coords3):
    mesh = plsc.VectorSubcoreMesh(core_axis_name="c", subcore_axis_name="s")
    params = pltpu.CompilerParams(
        needs_layout_passes=False, use_tc_tiling_on_sc=False
    )
    table = pl.kernel(
        _tr_body,
        out_type=jax.ShapeDtypeStruct((H * W, C), _F),
        mesh=mesh,
        scratch_types=[
            pltpu.VMEM((C, 8, 128), _F),
            pltpu.VMEM((C, 8, 128), _F),
            pltpu.VMEM((1024, C), _F),
            pltpu.VMEM((1024, C), _F),
            pltpu.SemaphoreType.DMA,
            pltpu.SemaphoreType.DMA,
            pltpu.SemaphoreType.DMA,
            pltpu.SemaphoreType.DMA,
        ],
        compiler_params=params,
    )(grid5)

    return pl.kernel(
        _sa_body,
        out_type=jax.ShapeDtypeStruct((2, NBLK_ALL, 8, BP), _F),
        mesh=mesh,
        scratch_types=[
            pltpu.VMEM((2, BP), _F),
            pltpu.VMEM((2, BP), _F),
            pltpu.VMEM((BP,), _I),
            pltpu.VMEM((BP,), _I),
            pltpu.VMEM((BP,), _I),
            pltpu.VMEM((BP,), _I),
            pltpu.VMEM((BP,), _I),
            pltpu.VMEM((BP,), _I),
            pltpu.VMEM((BP,), _I),
            pltpu.VMEM((BP,), _I),
            pltpu.VMEM((BP,), _F),
            pltpu.VMEM((BP,), _F),
            pltpu.VMEM((BP,), _F),
            pltpu.VMEM((BP,), _F),
            pltpu.VMEM((BP, C), _F),
            pltpu.VMEM((BP, C), _F),
            pltpu.VMEM((BP, C), _F),
            pltpu.VMEM((BP, C), _F),
            pltpu.VMEM((BP, C), _F),
            pltpu.VMEM((BP, C), _F),
            pltpu.VMEM((BP, C), _F),
            pltpu.VMEM((BP, C), _F),
            pltpu.VMEM((C, BP), _F),
            pltpu.VMEM((C, BP), _F),
            pltpu.SemaphoreType.DMA,
            pltpu.SemaphoreType.DMA,
            pltpu.SemaphoreType.DMA,
            pltpu.SemaphoreType.DMA,
            pltpu.SemaphoreType.DMA,
            pltpu.SemaphoreType.DMA,
        ],
        compiler_params=params,
    )(table, coords3)


def kernel(coords_local_xy, w_grid):
    coords = coords_local_xy.astype(jnp.float32)
    grid5 = jnp.transpose(
        jnp.reshape(w_grid[0], (C, TY, 8, TX, 128)), (0, 1, 3, 2, 4)
    )
    coords3 = jnp.transpose(
        jnp.reshape(coords, (NBLK_ALL, BP, 2)), (0, 2, 1)
    )
    v = _grid_sample_sc(grid5, coords3)
    return jnp.reshape(jnp.transpose(v, (1, 3, 0, 2)), (M, C))


# speedup vs baseline: 1.4405x; 1.0135x over previous
import jax
import jax.numpy as jnp
from jax import lax
from jax.experimental import pallas as pl
from jax.experimental.pallas import tpu as pltpu
from jax.experimental.pallas import tpu_sc as plsc

H = 2048
W = 2048
C = 16
M = 1048576

NC = 2
NS = 16
NW = NC * NS

_F = jnp.float32
_I = jnp.int32

TY = H // 8
TX = W // 128
NT = TY * TX
TPW = NT // NW

BP = 128
CHUNK = M // NW
NBK = CHUNK // BP
GPB = BP // 16
NBLK_ALL = M // BP


def _tr_body(grid_hbm, table_hbm, ch0, ch1, ot0, ot1, si0, si1, so0, so1):
    chs = (ch0, ch1)
    ots = (ot0, ot1)
    sis = (si0, si1)
    sos = (so0, so1)
    wid = lax.axis_index("s") * NC + lax.axis_index("c")
    base = wid * TPW
    iota = lax.iota(_I, 16)
    cvecs = [jnp.full((16,), c, dtype=_I) for c in range(C)]

    def fire_in(par, b):
        t = base + jnp.minimum(b, TPW - 1)
        ty = t // TX
        tx = t - ty * TX
        pltpu.async_copy(grid_hbm.at[:, ty, tx], chs[par], sis[par])

    def wait_in(par):
        pltpu.make_async_copy(grid_hbm.at[:, 0, 0], chs[par], sis[par]).wait()

    def interleave(par):
        def grp(k):
            iy = k // 8
            j = k - iy * 8
            iyv = jnp.broadcast_to(iy, (16,)).astype(_I)
            ixv = j * 16 + iota
            for c in range(C):
                v = chs[par][c, iy, pl.ds(j * 16, 16)]
                plsc.store_scatter(ots[par], [iyv, ixv, cvecs[c]], v)

        plsc.parallel_loop(0, 64, 1, unroll=4)(grp)

    def fire_out(par, b):
        t = base + b
        ty = t // TX
        tx = t - ty * TX
        pltpu.async_copy(ots[par], table_hbm.at[ty, :, tx], sos[par])

    def wait_out(par):
        pltpu.make_async_copy(ots[par], table_hbm.at[0, :, 0],
                              sos[par]).wait()

    fire_in(0, 0)

    def sup(it, carry):
        for par in (0, 1):
            b = 2 * it + par
            wait_in(par)
            fire_in(1 - par, b + 1)

            @pl.when(b >= 2)
            def _():
                wait_out(par)

            interleave(par)
            fire_out(par, b)
        return carry

    lax.fori_loop(0, TPW // 2, sup, 0)
    wait_in(0)
    wait_out(0)
    wait_out(1)


def _sa_body(table_hbm, coords_hbm, out_hbm,
             co0, co1,
             a0, b0, c0, d0, a1, b1, c1, d1,
             wx0, wy0, wx1, wy1,
             r00_0, r01_0, r10_0, r11_0, r00_1, r01_1, r10_1, r11_1,
             o0, o1,
             sc0, sc1, sg0, sg1, so0, so1):
    cos = (co0, co1)
    idxs = ((a0, b0, c0, d0), (a1, b1, c1, d1))
    ws = ((wx0, wy0), (wx1, wy1))
    rows = ((r00_0, r01_0, r10_0, r11_0), (r00_1, r01_1, r10_1, r11_1))
    outs = (o0, o1)
    scs = (sc0, sc1)
    sgs = (sg0, sg1)
    sos = (so0, so1)

    wid = lax.axis_index("s") * NC + lax.axis_index("c")
    bbase = wid * NBK
    iota = lax.iota(_I, 16)
    cdiags = [(iota + c) % 16 for c in range(C)]

    def fire_co(par, g):
        bid = bbase + jnp.minimum(g, NBK - 1)
        pltpu.async_copy(coords_hbm.at[bid], cos[par], scs[par])

    def wait_co(par):
        pltpu.make_async_copy(coords_hbm.at[0], cos[par], scs[par]).wait()

    def pass1(par):
        i00_v, i01_v, i10_v, i11_v = idxs[par]
        wx_v, wy_v = ws[par]

        def grp(j):
            sl = pl.ds(j * 16, 16)
            x = cos[par][0, sl]
            y = cos[par][1, sl]
            u = 2.0 * x - 1.0
            v = 2.0 * y - 1.0
            xp = (u + 1.0) * 0.5 * (W - 1)
            yp = (v + 1.0) * 0.5 * (H - 1)
            xp = jnp.minimum(jnp.maximum(xp, 0.0), float(W - 1))
            yp = jnp.minimum(jnp.maximum(yp, 0.0), float(H - 1))
            x0i = xp.astype(_I)
            y0i = yp.astype(_I)
            wx_v[sl] = xp - x0i.astype(_F)
            wy_v[sl] = yp - y0i.astype(_F)
            x1i = jnp.minimum(x0i + 1, W - 1)
            y1i = jnp.minimum(y0i + 1, H - 1)
            y0w = y0i * W
            y1w = y1i * W
            i00_v[sl] = y0w + x0i
            i01_v[sl] = y0w + x1i
            i10_v[sl] = y1w + x0i
            i11_v[sl] = y1w + x1i

        plsc.parallel_loop(0, GPB, 1, unroll=GPB)(grp)

    def fire_g(par):
        for i in range(4):
            pltpu.async_copy(table_hbm.at[idxs[par][i]], rows[par][i],
                             sgs[par])

    def wait_g(par):
        for i in range(4):
            pltpu.make_async_copy(table_hbm.at[idxs[par][i]], rows[par][i],
                                  sgs[par]).wait()

    def pass2(par):
        r00_v, r01_v, r10_v, r11_v = rows[par]
        wx_v, wy_v = ws[par]

        def grp(j):
            sl = pl.ds(j * 16, 16)
            pidx = j * 16 + iota
            wx = wx_v[sl]
            wy = wy_v[sl]
            w11 = wx * wy
            w10 = wy - w11
            w01 = wx - w11
            w00 = (1.0 - wx) - w10
            for c in range(C):
                cd = cdiags[c]
                ia = plsc.load_gather(r00_v, [pidx, cd])
                ib = plsc.load_gather(r01_v, [pidx, cd])
                ic = plsc.load_gather(r10_v, [pidx, cd])
                id_ = plsc.load_gather(r11_v, [pidx, cd])
                val = ia * w00 + ib * w01 + ic * w10 + id_ * w11
                plsc.store_scatter(outs[par], [cd, pidx], val)

        plsc.parallel_loop(0, GPB, 1, unroll=4)(grp)

    def fire_o(par, g):
        bid = bbase + g
        pltpu.async_copy(outs[par].at[pl.ds(0, 8)], out_hbm.at[0, bid],
                         sos[par])
        pltpu.async_copy(outs[par].at[pl.ds(8, 8)], out_hbm.at[1, bid],
                         sos[par])

    def wait_o(par):
        pltpu.make_async_copy(outs[par].at[pl.ds(0, 8)], out_hbm.at[0, 0],
                              sos[par]).wait()
        pltpu.make_async_copy(outs[par].at[pl.ds(8, 8)], out_hbm.at[1, 0],
                              sos[par]).wait()

    fire_co(0, 0)
    wait_co(0)
    pass1(0)
    fire_g(0)
    fire_co(1, 1)

    def sup(it, carry):
        for par in (0, 1):
            g = 2 * it + par
            wait_g(par)
            wait_co(1 - par)
            pass1(1 - par)
            fire_g(1 - par)
            fire_co(par, g + 2)

            @pl.when(g >= 2)
            def _():
                wait_o(par)

            pass2(par)
            fire_o(par, g)
        return carry

    lax.fori_loop(0, NBK // 2, sup, 0)
    wait_g(0)
    wait_co(1)
    wait_o(0)
    wait_o(1)


@jax.jit
def _grid_sample_sc(grid5, coords3):
    mesh = plsc.VectorSubcoreMesh(core_axis_name="c", subcore_axis_name="s")
    params = pltpu.CompilerParams(
        needs_layout_passes=False, use_tc_tiling_on_sc=False
    )
    table5 = pl.kernel(
        _tr_body,
        out_type=jax.ShapeDtypeStruct((TY, 8, TX, 128, C), _F),
        mesh=mesh,
        scratch_types=[
            pltpu.VMEM((C, 8, 128), _F),
            pltpu.VMEM((C, 8, 128), _F),
            pltpu.VMEM((8, 128, C), _F),
            pltpu.VMEM((8, 128, C), _F),
            pltpu.SemaphoreType.DMA,
            pltpu.SemaphoreType.DMA,
            pltpu.SemaphoreType.DMA,
            pltpu.SemaphoreType.DMA,
        ],
        compiler_params=params,
    )(grid5)
    table = jnp.reshape(table5, (H * W, C))

    return pl.kernel(
        _sa_body,
        out_type=jax.ShapeDtypeStruct((2, NBLK_ALL, 8, BP), _F),
        mesh=mesh,
        scratch_types=[
            pltpu.VMEM((2, BP), _F),
            pltpu.VMEM((2, BP), _F),
            pltpu.VMEM((BP,), _I),
            pltpu.VMEM((BP,), _I),
            pltpu.VMEM((BP,), _I),
            pltpu.VMEM((BP,), _I),
            pltpu.VMEM((BP,), _I),
            pltpu.VMEM((BP,), _I),
            pltpu.VMEM((BP,), _I),
            pltpu.VMEM((BP,), _I),
            pltpu.VMEM((BP,), _F),
            pltpu.VMEM((BP,), _F),
            pltpu.VMEM((BP,), _F),
            pltpu.VMEM((BP,), _F),
            pltpu.VMEM((BP, C), _F),
            pltpu.VMEM((BP, C), _F),
            pltpu.VMEM((BP, C), _F),
            pltpu.VMEM((BP, C), _F),
            pltpu.VMEM((BP, C), _F),
            pltpu.VMEM((BP, C), _F),
            pltpu.VMEM((BP, C), _F),
            pltpu.VMEM((BP, C), _F),
            pltpu.VMEM((C, BP), _F),
            pltpu.VMEM((C, BP), _F),
            pltpu.SemaphoreType.DMA,
            pltpu.SemaphoreType.DMA,
            pltpu.SemaphoreType.DMA,
            pltpu.SemaphoreType.DMA,
            pltpu.SemaphoreType.DMA,
            pltpu.SemaphoreType.DMA,
        ],
        compiler_params=params,
    )(table, coords3)


def kernel(coords_local_xy, w_grid):
    coords = coords_local_xy.astype(jnp.float32)
    grid5 = jnp.transpose(
        jnp.reshape(w_grid[0], (C, TY, 8, TX, 128)), (0, 1, 3, 2, 4)
    )
    coords3 = jnp.transpose(
        jnp.reshape(coords, (NBLK_ALL, BP, 2)), (0, 2, 1)
    )
    v = _grid_sample_sc(grid5, coords3)
    return jnp.reshape(jnp.transpose(v, (1, 3, 0, 2)), (M, C))


# speedup vs baseline: 1.8171x; 1.2614x over previous
import jax
import jax.numpy as jnp
from jax import lax
from jax.experimental import pallas as pl
from jax.experimental.pallas import tpu as pltpu
from jax.experimental.pallas import tpu_sc as plsc

H = 2048
W = 2048
C = 16
M = 1048576

NC = 2
NS = 16
NW = NC * NS

_F = jnp.float32
_I = jnp.int32

TY = H // 8
TX = W // 128
NT = TY * TX
TPW = NT // NW

BP = 128
CHUNK = M // NW
NBK = CHUNK // BP
GPB = BP // 16
NBLK_ALL = M // BP


def _tr_body(grid_hbm, table_hbm, ch0, ch1, ot0, ot1, si0, si1, so0, so1):
    chs = (ch0, ch1)
    ots = (ot0, ot1)
    sis = (si0, si1)
    sos = (so0, so1)
    wid = lax.axis_index("s") * NC + lax.axis_index("c")
    base = wid * TPW
    iota = lax.iota(_I, 16)
    cdiags = [(iota + d) % 16 for d in range(C)]

    def fire_in(par, b):
        t = base + jnp.minimum(b, TPW - 1)
        ty = t // TX
        tx = t - ty * TX
        pltpu.async_copy(grid_hbm.at[:, ty, tx], chs[par], sis[par])

    def wait_in(par):
        pltpu.make_async_copy(grid_hbm.at[:, 0, 0], chs[par], sis[par]).wait()

    def interleave(par):
        def grp(k):
            iy = k // 8
            j = k - iy * 8
            iyv = jnp.broadcast_to(iy, (16,)).astype(_I)
            ixv = j * 16 + iota
            for d in range(C):
                cd = cdiags[d]
                v = plsc.load_gather(chs[par], [cd, iyv, ixv])
                plsc.store_scatter(ots[par], [iyv, ixv, cd], v)

        plsc.parallel_loop(0, 64, 1, unroll=4)(grp)

    def fire_out(par, b):
        t = base + b
        ty = t // TX
        tx = t - ty * TX
        pltpu.async_copy(ots[par], table_hbm.at[ty, :, tx], sos[par])

    def wait_out(par):
        pltpu.make_async_copy(ots[par], table_hbm.at[0, :, 0],
                              sos[par]).wait()

    fire_in(0, 0)

    def sup(it, carry):
        for par in (0, 1):
            b = 2 * it + par
            wait_in(par)
            fire_in(1 - par, b + 1)

            @pl.when(b >= 2)
            def _():
                wait_out(par)

            interleave(par)
            fire_out(par, b)
        return carry

    lax.fori_loop(0, TPW // 2, sup, 0)
    wait_in(0)
    wait_out(0)
    wait_out(1)


def _sa_body(table_hbm, coords_hbm, out_hbm,
             co0, co1,
             a0, b0, c0, d0, a1, b1, c1, d1,
             wx0, wy0, wx1, wy1,
             r00_0, r01_0, r10_0, r11_0, r00_1, r01_1, r10_1, r11_1,
             o0, o1,
             sc0, sc1, sg0, sg1, so0, so1):
    cos = (co0, co1)
    idxs = ((a0, b0, c0, d0), (a1, b1, c1, d1))
    ws = ((wx0, wy0), (wx1, wy1))
    rows = ((r00_0, r01_0, r10_0, r11_0), (r00_1, r01_1, r10_1, r11_1))
    outs = (o0, o1)
    scs = (sc0, sc1)
    sgs = (sg0, sg1)
    sos = (so0, so1)

    wid = lax.axis_index("s") * NC + lax.axis_index("c")
    bbase = wid * NBK
    iota = lax.iota(_I, 16)
    cdiags = [(iota + c) % 16 for c in range(C)]

    def fire_co(par, g):
        bid = bbase + jnp.minimum(g, NBK - 1)
        pltpu.async_copy(coords_hbm.at[bid], cos[par], scs[par])

    def wait_co(par):
        pltpu.make_async_copy(coords_hbm.at[0], cos[par], scs[par]).wait()

    def pass1(par):
        i00_v, i01_v, i10_v, i11_v = idxs[par]
        wx_v, wy_v = ws[par]

        def grp(j):
            sl = pl.ds(j * 16, 16)
            x = cos[par][0, sl]
            y = cos[par][1, sl]
            u = 2.0 * x - 1.0
            v = 2.0 * y - 1.0
            xp = (u + 1.0) * 0.5 * (W - 1)
            yp = (v + 1.0) * 0.5 * (H - 1)
            xp = jnp.minimum(jnp.maximum(xp, 0.0), float(W - 1))
            yp = jnp.minimum(jnp.maximum(yp, 0.0), float(H - 1))
            x0i = xp.astype(_I)
            y0i = yp.astype(_I)
            wx_v[sl] = xp - x0i.astype(_F)
            wy_v[sl] = yp - y0i.astype(_F)
            x1i = jnp.minimum(x0i + 1, W - 1)
            y1i = jnp.minimum(y0i + 1, H - 1)
            y0w = y0i * W
            y1w = y1i * W
            i00_v[sl] = y0w + x0i
            i01_v[sl] = y0w + x1i
            i10_v[sl] = y1w + x0i
            i11_v[sl] = y1w + x1i

        plsc.parallel_loop(0, GPB, 1, unroll=GPB)(grp)

    def fire_g(par):
        for i in range(4):
            pltpu.async_copy(table_hbm.at[idxs[par][i]], rows[par][i],
                             sgs[par])

    def wait_g(par):
        for i in range(4):
            pltpu.make_async_copy(table_hbm.at[idxs[par][i]], rows[par][i],
                                  sgs[par]).wait()

    def pass2(par):
        r00_v, r01_v, r10_v, r11_v = rows[par]
        wx_v, wy_v = ws[par]

        def grp(j):
            sl = pl.ds(j * 16, 16)
            pidx = j * 16 + iota
            wx = wx_v[sl]
            wy = wy_v[sl]
            w11 = wx * wy
            w10 = wy - w11
            w01 = wx - w11
            w00 = (1.0 - wx) - w10
            for c in range(C):
                cd = cdiags[c]
                ia = plsc.load_gather(r00_v, [pidx, cd])
                ib = plsc.load_gather(r01_v, [pidx, cd])
                ic = plsc.load_gather(r10_v, [pidx, cd])
                id_ = plsc.load_gather(r11_v, [pidx, cd])
                val = ia * w00 + ib * w01 + ic * w10 + id_ * w11
                plsc.store_scatter(outs[par], [cd, pidx], val)

        plsc.parallel_loop(0, GPB, 1, unroll=4)(grp)

    def fire_o(par, g):
        bid = bbase + g
        pltpu.async_copy(outs[par].at[pl.ds(0, 8)], out_hbm.at[0, bid],
                         sos[par])
        pltpu.async_copy(outs[par].at[pl.ds(8, 8)], out_hbm.at[1, bid],
                         sos[par])

    def wait_o(par):
        pltpu.make_async_copy(outs[par].at[pl.ds(0, 8)], out_hbm.at[0, 0],
                              sos[par]).wait()
        pltpu.make_async_copy(outs[par].at[pl.ds(8, 8)], out_hbm.at[1, 0],
                              sos[par]).wait()

    fire_co(0, 0)
    wait_co(0)
    pass1(0)
    fire_g(0)
    fire_co(1, 1)

    def sup(it, carry):
        for par in (0, 1):
            g = 2 * it + par
            wait_g(par)
            wait_co(1 - par)
            pass1(1 - par)
            fire_g(1 - par)
            fire_co(par, g + 2)

            @pl.when(g >= 2)
            def _():
                wait_o(par)

            pass2(par)
            fire_o(par, g)
        return carry

    lax.fori_loop(0, NBK // 2, sup, 0)
    wait_g(0)
    wait_co(1)
    wait_o(0)
    wait_o(1)


@jax.jit
def _grid_sample_sc(grid5, coords3):
    mesh = plsc.VectorSubcoreMesh(core_axis_name="c", subcore_axis_name="s")
    params = pltpu.CompilerParams(
        needs_layout_passes=False, use_tc_tiling_on_sc=False
    )
    table5 = pl.kernel(
        _tr_body,
        out_type=jax.ShapeDtypeStruct((TY, 8, TX, 128, C), _F),
        mesh=mesh,
        scratch_types=[
            pltpu.VMEM((C, 8, 128), _F),
            pltpu.VMEM((C, 8, 128), _F),
            pltpu.VMEM((8, 128, C), _F),
            pltpu.VMEM((8, 128, C), _F),
            pltpu.SemaphoreType.DMA,
            pltpu.SemaphoreType.DMA,
            pltpu.SemaphoreType.DMA,
            pltpu.SemaphoreType.DMA,
        ],
        compiler_params=params,
    )(grid5)
    table = jnp.reshape(table5, (H * W, C))

    return pl.kernel(
        _sa_body,
        out_type=jax.ShapeDtypeStruct((2, NBLK_ALL, 8, BP), _F),
        mesh=mesh,
        scratch_types=[
            pltpu.VMEM((2, BP), _F),
            pltpu.VMEM((2, BP), _F),
            pltpu.VMEM((BP,), _I),
            pltpu.VMEM((BP,), _I),
            pltpu.VMEM((BP,), _I),
            pltpu.VMEM((BP,), _I),
            pltpu.VMEM((BP,), _I),
            pltpu.VMEM((BP,), _I),
            pltpu.VMEM((BP,), _I),
            pltpu.VMEM((BP,), _I),
            pltpu.VMEM((BP,), _F),
            pltpu.VMEM((BP,), _F),
            pltpu.VMEM((BP,), _F),
            pltpu.VMEM((BP,), _F),
            pltpu.VMEM((BP, C), _F),
            pltpu.VMEM((BP, C), _F),
            pltpu.VMEM((BP, C), _F),
            pltpu.VMEM((BP, C), _F),
            pltpu.VMEM((BP, C), _F),
            pltpu.VMEM((BP, C), _F),
            pltpu.VMEM((BP, C), _F),
            pltpu.VMEM((BP, C), _F),
            pltpu.VMEM((C, BP), _F),
            pltpu.VMEM((C, BP), _F),
            pltpu.SemaphoreType.DMA,
            pltpu.SemaphoreType.DMA,
            pltpu.SemaphoreType.DMA,
            pltpu.SemaphoreType.DMA,
            pltpu.SemaphoreType.DMA,
            pltpu.SemaphoreType.DMA,
        ],
        compiler_params=params,
    )(table, coords3)


def kernel(coords_local_xy, w_grid):
    coords = coords_local_xy.astype(jnp.float32)
    grid5 = jnp.transpose(
        jnp.reshape(w_grid[0], (C, TY, 8, TX, 128)), (0, 1, 3, 2, 4)
    )
    coords3 = jnp.transpose(
        jnp.reshape(coords, (NBLK_ALL, BP, 2)), (0, 2, 1)
    )
    v = _grid_sample_sc(grid5, coords3)
    return jnp.reshape(jnp.transpose(v, (1, 3, 0, 2)), (M, C))
